# baseline (speedup 1.0000x reference)
import functools
import jax
import jax.numpy as jnp
from jax import lax
from jax.experimental import pallas as pl
from jax.experimental.pallas import tpu as pltpu
from jax.experimental.pallas import tpu_sc as plsc

B = 16384
D = 128
NC = 2
NS = 16
NW = NC * NS
RPW = B // NW
CH = 128
NCHUNK = RPW // CH


def _gmf_body(uid_hbm, iid_hbm, ut_hbm, it_hbm, wb_hbm, out_hbm,
              uid_v, iid_v, u_rows, i_rows, wb_v, out_v, sem_u, sem_i):
    wid = lax.axis_index("s") * NC + lax.axis_index("c")
    pltpu.sync_copy(wb_hbm, wb_v)
    pltpu.sync_copy(uid_hbm.at[wid], uid_v)
    pltpu.sync_copy(iid_hbm.at[wid], iid_v)
    lanes = lax.iota(jnp.int32, 16)
    ngroups = CH // 16
    b_bc = plsc.load_gather(wb_v, [jnp.full((16,), D, jnp.int32)])
    for g in range(NCHUNK):
        cu = pltpu.async_copy(ut_hbm.at[uid_v.at[g]], u_rows, sem_u)
        ci = pltpu.async_copy(it_hbm.at[iid_v.at[g]], i_rows, sem_i)
        cu.wait()
        ci.wait()

        def col_body(d, accs):
            dv = jnp.full((16,), d, jnp.int32)
            w = plsc.load_gather(wb_v, [dv])
            new = []
            for gr in range(ngroups):
                rows = gr * 16 + lanes
                u = plsc.load_gather(u_rows, [rows, dv])
                v = plsc.load_gather(i_rows, [rows, dv])
                new.append(accs[gr] + u * v * w)
            return tuple(new)

        accs = lax.fori_loop(
            0, D, col_body,
            tuple(jnp.zeros((16,), jnp.float32) for _ in range(ngroups)))
        for gr in range(ngroups):
            out_v[pl.ds(g * CH + gr * 16, 16)] = accs[gr] + b_bc
    pltpu.sync_copy(out_v, out_hbm.at[pl.ds(wid * RPW, RPW)])


@jax.jit
def kernel(user_id, item_id, user_table, item_table, W, b):
    wb = jnp.concatenate([W.reshape(-1), b, jnp.zeros((7,), jnp.float32)])
    uid = user_id.astype(jnp.int32).reshape(NW, NCHUNK, CH)
    iid = item_id.astype(jnp.int32).reshape(NW, NCHUNK, CH)
    mesh = plsc.VectorSubcoreMesh(core_axis_name="c", subcore_axis_name="s")
    run = functools.partial(
        pl.kernel,
        mesh=mesh,
        out_type=jax.ShapeDtypeStruct((B,), jnp.float32),
        scratch_types=[
            pltpu.VMEM((NCHUNK, CH), jnp.int32),
            pltpu.VMEM((NCHUNK, CH), jnp.int32),
            pltpu.VMEM((CH, D), jnp.float32),
            pltpu.VMEM((CH, D), jnp.float32),
            pltpu.VMEM((D + 8,), jnp.float32),
            pltpu.VMEM((RPW,), jnp.float32),
            pltpu.SemaphoreType.DMA,
            pltpu.SemaphoreType.DMA,
        ],
        compiler_params=pltpu.CompilerParams(needs_layout_passes=False),
    )(_gmf_body)
    return run(uid, iid, user_table, item_table, wb)


# speedup vs baseline: 2.6410x; 2.6410x over previous
import functools
import jax
import jax.numpy as jnp
from jax import lax
from jax.experimental import pallas as pl
from jax.experimental.pallas import tpu as pltpu
from jax.experimental.pallas import tpu_sc as plsc

B = 16384
D = 128
NC = 2
NS = 16
NW = NC * NS
RPW = B // NW
CH = 128
NCHUNK = RPW // CH


def _gmf_body(uid_hbm, iid_hbm, ut_hbm, it_hbm, wb_hbm, out_hbm,
              uid_v, iid_v, u_rows, i_rows, wb_v, out_v, sem_u, sem_i):
    wid = lax.axis_index("s") * NC + lax.axis_index("c")
    pltpu.sync_copy(wb_hbm, wb_v)
    pltpu.sync_copy(uid_hbm.at[wid], uid_v)
    pltpu.sync_copy(iid_hbm.at[wid], iid_v)
    lanes = lax.iota(jnp.int32, 16)
    ngroups = CH // 16
    b_bc = plsc.load_gather(wb_v, [jnp.full((16,), D, jnp.int32)])
    for g in range(NCHUNK):
        cu = pltpu.async_copy(ut_hbm.at[uid_v.at[g]], u_rows, sem_u)
        ci = pltpu.async_copy(it_hbm.at[iid_v.at[g]], i_rows, sem_i)
        cu.wait()
        ci.wait()

        def col_body(d, accs):
            cols = (d + lanes) & (D - 1)
            w = plsc.load_gather(wb_v, [cols])
            new = []
            for gr in range(ngroups):
                rows = gr * 16 + lanes
                u = plsc.load_gather(u_rows, [rows, cols])
                v = plsc.load_gather(i_rows, [rows, cols])
                new.append(accs[gr] + u * v * w)
            return tuple(new)

        accs0 = tuple(jnp.zeros((16,), jnp.float32) for _ in range(ngroups))
        accs = plsc.parallel_loop(0, D, 1, unroll=4, carry=accs0)(col_body)
        for gr in range(ngroups):
            out_v[pl.ds(g * CH + gr * 16, 16)] = accs[gr] + b_bc
    pltpu.sync_copy(out_v, out_hbm.at[pl.ds(wid * RPW, RPW)])


@jax.jit
def kernel(user_id, item_id, user_table, item_table, W, b):
    wb = jnp.concatenate([W.reshape(-1), b, jnp.zeros((7,), jnp.float32)])
    uid = user_id.astype(jnp.int32).reshape(NW, NCHUNK, CH)
    iid = item_id.astype(jnp.int32).reshape(NW, NCHUNK, CH)
    mesh = plsc.VectorSubcoreMesh(core_axis_name="c", subcore_axis_name="s")
    run = functools.partial(
        pl.kernel,
        mesh=mesh,
        out_type=jax.ShapeDtypeStruct((B,), jnp.float32),
        scratch_types=[
            pltpu.VMEM((NCHUNK, CH), jnp.int32),
            pltpu.VMEM((NCHUNK, CH), jnp.int32),
            pltpu.VMEM((CH, D), jnp.float32),
            pltpu.VMEM((CH, D), jnp.float32),
            pltpu.VMEM((D + 8,), jnp.float32),
            pltpu.VMEM((RPW,), jnp.float32),
            pltpu.SemaphoreType.DMA,
            pltpu.SemaphoreType.DMA,
        ],
        compiler_params=pltpu.CompilerParams(needs_layout_passes=False),
    )(_gmf_body)
    return run(uid, iid, user_table, item_table, wb)


# speedup vs baseline: 3.1467x; 1.1915x over previous
import functools
import jax
import jax.numpy as jnp
from jax import lax
from jax.experimental import pallas as pl
from jax.experimental.pallas import tpu as pltpu
from jax.experimental.pallas import tpu_sc as plsc

B = 16384
D = 128
NC = 2
NS = 16
NW = NC * NS
RPW = B // NW
CH = 128
NCHUNK = RPW // CH


def _gmf_body(uid_hbm, iid_hbm, ut_hbm, it_hbm, wb_hbm, out_hbm,
              uid_v, iid_v, u_rows0, u_rows1, i_rows0, i_rows1, wb_v, out_v,
              sem_u0, sem_u1, sem_i0, sem_i1):
    wid = lax.axis_index("s") * NC + lax.axis_index("c")
    pltpu.sync_copy(uid_hbm.at[wid], uid_v)
    pltpu.sync_copy(iid_hbm.at[wid], iid_v)
    u_bufs = (u_rows0, u_rows1)
    i_bufs = (i_rows0, i_rows1)
    sems_u = (sem_u0, sem_u1)
    sems_i = (sem_i0, sem_i1)

    def issue(g):
        s = g & 1
        cu = pltpu.async_copy(ut_hbm.at[uid_v.at[g]], u_bufs[s], sems_u[s])
        ci = pltpu.async_copy(it_hbm.at[iid_v.at[g]], i_bufs[s], sems_i[s])
        return cu, ci

    inflight = issue(0)
    pltpu.sync_copy(wb_hbm, wb_v)
    lanes = lax.iota(jnp.int32, 16)
    ngroups = CH // 16
    b_bc = plsc.load_gather(wb_v, [jnp.full((16,), D, jnp.int32)])
    for g in range(NCHUNK):
        cu, ci = inflight
        cu.wait()
        ci.wait()
        if g + 1 < NCHUNK:
            inflight = issue(g + 1)
        s = g & 1
        u_rows = u_bufs[s]
        i_rows = i_bufs[s]

        def col_body(d, accs):
            cols = (d + lanes) & (D - 1)
            w = plsc.load_gather(wb_v, [cols])
            new = []
            for gr in range(ngroups):
                rows = gr * 16 + lanes
                u = plsc.load_gather(u_rows, [rows, cols])
                v = plsc.load_gather(i_rows, [rows, cols])
                new.append(accs[gr] + u * v * w)
            return tuple(new)

        accs0 = tuple(jnp.zeros((16,), jnp.float32) for _ in range(ngroups))
        accs = plsc.parallel_loop(0, D, 1, unroll=4, carry=accs0)(col_body)
        for gr in range(ngroups):
            out_v[pl.ds(g * CH + gr * 16, 16)] = accs[gr] + b_bc
    pltpu.sync_copy(out_v, out_hbm.at[pl.ds(wid * RPW, RPW)])


@jax.jit
def kernel(user_id, item_id, user_table, item_table, W, b):
    wb = jnp.concatenate([W.reshape(-1), b, jnp.zeros((7,), jnp.float32)])
    uid = user_id.astype(jnp.int32).reshape(NW, NCHUNK, CH)
    iid = item_id.astype(jnp.int32).reshape(NW, NCHUNK, CH)
    mesh = plsc.VectorSubcoreMesh(core_axis_name="c", subcore_axis_name="s")
    run = functools.partial(
        pl.kernel,
        mesh=mesh,
        out_type=jax.ShapeDtypeStruct((B,), jnp.float32),
        scratch_types=[
            pltpu.VMEM((NCHUNK, CH), jnp.int32),
            pltpu.VMEM((NCHUNK, CH), jnp.int32),
            pltpu.VMEM((CH, D), jnp.float32),
            pltpu.VMEM((CH, D), jnp.float32),
            pltpu.VMEM((CH, D), jnp.float32),
            pltpu.VMEM((CH, D), jnp.float32),
            pltpu.VMEM((D + 8,), jnp.float32),
            pltpu.VMEM((RPW,), jnp.float32),
            pltpu.SemaphoreType.DMA,
            pltpu.SemaphoreType.DMA,
            pltpu.SemaphoreType.DMA,
            pltpu.SemaphoreType.DMA,
        ],
        compiler_params=pltpu.CompilerParams(needs_layout_passes=False),
    )(_gmf_body)
    return run(uid, iid, user_table, item_table, wb)
